# baseline (speedup 1.0000x reference)
import functools

import jax
import jax.numpy as jnp
from jax import lax
from jax.experimental import pallas as pl
from jax.experimental.pallas import tpu as pltpu
from jax.experimental.pallas import tpu_sc as plsc

CH = 128
KB = 8


def _sc_scatter(prev_pad, srcs2, dsts2, n_nodes, d, nc, ns, n_super):
  nw = nc * ns
  blk = 80
  nblocks = n_nodes // blk
  assert n_nodes % blk == 0 and blk % 16 == 0

  mesh = plsc.VectorSubcoreMesh(core_axis_name="c", subcore_axis_name="s")

  @functools.partial(
      pl.kernel,
      out_type=jax.ShapeDtypeStruct((nc * n_nodes, d), jnp.float32),
      mesh=mesh,
      scratch_types=[
          pltpu.VMEM((KB, CH), jnp.int32),
          pltpu.VMEM((KB, CH), jnp.int32),
          pltpu.VMEM((2, CH, d), jnp.float32),
          pltpu.VMEM((16, d), jnp.float32),
          pltpu.VMEM_SHARED((n_nodes, d), jnp.float32),
          pltpu.SemaphoreType.DMA,
          pltpu.SemaphoreType.DMA,
          pltpu.SemaphoreType.DMA,
          pltpu.SemaphoreType.DMA,
      ],
  )
  def body(prev_hbm, srcs_hbm, dsts_hbm, out_hbm, sidx_blk, didx_blk, rows,
           zbuf, acc, gsem0, gsem1, ssem0, ssem1):
    cid = lax.axis_index("c")
    sid = lax.axis_index("s")
    wid = sid * nc + cid
    gsems = (gsem0, gsem1)
    ssems = (ssem0, ssem1)
    my_nblk = (nblocks - 1 - sid) // ns + 1

    zvec = jnp.zeros((16,), jnp.float32)
    for r in range(16):
      for c in range(d // 16):
        zbuf[r, pl.ds(c * 16, 16)] = zvec

    def zero_body(j, carry):
      base = (sid + j * ns) * blk
      for k in range(blk // 16):
        pltpu.sync_copy(zbuf, acc.at[pl.ds(base + k * 16, 16)])
      return carry

    lax.fori_loop(0, my_nblk, zero_body, 0)
    plsc.subcore_barrier()

    def super_body(s, carry):
      crow = wid * (n_super * KB) + s * KB
      pltpu.sync_copy(srcs_hbm.at[pl.ds(crow, KB)], sidx_blk)
      pltpu.sync_copy(dsts_hbm.at[pl.ds(crow, KB)], didx_blk)
      gdesc = [None] * KB
      sdesc = [None] * KB
      for j in range(KB):
        if j >= 2:
          sdesc[j - 2].wait()
        gdesc[j] = pltpu.async_copy(
            prev_hbm.at[sidx_blk.at[j]], rows.at[j % 2], gsems[j % 2])
        if j >= 1:
          gdesc[j - 1].wait()
          sdesc[j - 1] = pltpu.async_copy(
              rows.at[(j - 1) % 2], acc.at[didx_blk.at[j - 1]],
              ssems[(j - 1) % 2], add=True)
      sdesc[KB - 2].wait()
      gdesc[KB - 1].wait()
      sdesc[KB - 1] = pltpu.async_copy(
          rows.at[(KB - 1) % 2], acc.at[didx_blk.at[KB - 1]],
          ssems[(KB - 1) % 2], add=True)
      sdesc[KB - 1].wait()
      return carry

    lax.fori_loop(0, n_super, super_body, 0)
    plsc.subcore_barrier()

    def wr_body(j, carry):
      base = (sid + j * ns) * blk
      pltpu.sync_copy(acc.at[pl.ds(base, blk)],
                      out_hbm.at[pl.ds(cid * n_nodes + base, blk)])
      return carry

    lax.fori_loop(0, my_nblk, wr_body, 0)

  return body(prev_pad, srcs2, dsts2)


def _tc_finish(p0, p1, nodef, edgef, w2, n_nodes, d):
  bn = 400
  grid = n_nodes // bn

  def body(p0_ref, p1_ref, nf_ref, ef_ref, w2_ref, out_ref):
    nbr = p0_ref[...] + p1_ref[...]
    x2 = lax.dot_general(
        nbr, w2_ref[...],
        dimension_numbers=(((1,), (1,)), ((), ())),
        preferred_element_type=jnp.float32,
    )
    x = nf_ref[...] + ef_ref[...] + x2
    out_ref[...] = jnp.where(x >= 0, x, 0.01 * x)

  row_spec = pl.BlockSpec((bn, d), lambda i: (i, 0))
  return pl.pallas_call(
      body,
      grid=(grid,),
      in_specs=[row_spec, row_spec, row_spec, row_spec,
                pl.BlockSpec((d, d), lambda i: (0, 0))],
      out_specs=row_spec,
      out_shape=jax.ShapeDtypeStruct((n_nodes, d), jnp.float32),
  )(p0, p1, nodef, edgef, w2)


def kernel(prev_embeddings, edges_ij, node_features_embeddings, edge_features_embeddings, W2):
  b, n, d = prev_embeddings.shape
  e = edges_ij.shape[0]

  info = plsc.get_sparse_core_info()
  nc, ns = info.num_cores, info.num_subcores
  nw = nc * ns

  sc_bytes = CH * KB
  n_super = -(-2 * e // (nw * sc_bytes))
  pe = nw * n_super * sc_bytes

  u = edges_ij[:, 0]
  v = edges_ij[:, 1]
  pad = pe - 2 * e
  srcs = jnp.concatenate([v, u, jnp.full((pad,), n, jnp.int32)])
  dsts = jnp.concatenate([u, v, jnp.zeros((pad,), jnp.int32)])
  srcs2 = srcs.reshape(pe // CH, CH)
  dsts2 = dsts.reshape(pe // CH, CH)
  prev_pad = jnp.concatenate(
      [prev_embeddings[0], jnp.zeros((8, d), jnp.float32)], axis=0)

  partials = _sc_scatter(prev_pad, srcs2, dsts2, n, d, nc, ns, n_super)
  p0 = partials[:n]
  p1 = partials[n:]

  out = _tc_finish(p0, p1, node_features_embeddings[0],
                   edge_features_embeddings[0], W2, n, d)
  return out.reshape(b, n, d)
